# Initial kernel scaffold; baseline (speedup 1.0000x reference)
import jax
import jax.numpy as jnp
from jax.experimental import pallas as pl


def kernel(x, enc_w, enc_b, pre_w, pre_b, codebook, post_w, post_b, dec_w, dec_b):
    raise NotImplementedError("write your pallas kernel here")


# speedup vs baseline: 9.0177x; 9.0177x over previous
import functools

import jax
import jax.numpy as jnp
from jax.experimental import pallas as pl
from jax.experimental.pallas import tpu as pltpu

B, CIN, HW, P = 32, 3, 224, 16
ZC = 256
EMBED = 64
VOCAB = 8192
HL = HW // P
N = B * HL * HL
KENC = CIN * P * P
ROWS = 224

_DOT = jax.lax.Precision.DEFAULT
_EXACT = jax.lax.Precision.HIGHEST


def _body(p_ref, we_ref, eb_ref, wp_ref, pb_ref, cb_ref, cbt_ref,
          wpost_ref, postb_ref, wdec_ref, db_ref,
          z_ref, zq_ref, rec_ref):
    p = p_ref[...]
    z0 = jax.lax.dot_general(p, we_ref[...], (((1,), (0,)), ((), ())),
                             precision=_DOT) + eb_ref[...]
    z = jax.lax.dot_general(z0, wp_ref[...], (((1,), (0,)), ((), ())),
                            precision=_DOT) + pb_ref[...]
    z_ref[...] = z
    s1 = jnp.sum(z * z, axis=1, keepdims=True)
    cbt = cbt_ref[...]
    s2 = jnp.sum(cbt * cbt, axis=0, keepdims=True)
    m = jax.lax.dot_general(z, cbt, (((1,), (0,)), ((), ())),
                            precision=_DOT)
    dist = (s1 + s2) - 2.0 * m
    minval = jnp.min(dist, axis=1, keepdims=True)
    lane = jax.lax.broadcasted_iota(jnp.int32, dist.shape, 1)
    tok = jnp.min(jnp.where(dist == minval, lane, VOCAB), axis=1,
                  keepdims=True)
    oh = (lane == tok).astype(jnp.float32)
    zq = jax.lax.dot_general(oh, cb_ref[...], (((1,), (0,)), ((), ())),
                             precision=_EXACT)
    zq_ref[...] = zq
    d1 = jax.lax.dot_general(zq, wpost_ref[...], (((1,), (0,)), ((), ())),
                             precision=_DOT) + postb_ref[...]
    rec_ref[...] = jax.lax.dot_general(d1, wdec_ref[...],
                                       (((1,), (0,)), ((), ())),
                                       precision=_DOT) + db_ref[...]


@functools.partial(jax.jit, static_argnums=())
def kernel(x, enc_w, enc_b, pre_w, pre_b, codebook, post_w, post_b, dec_w, dec_b):
    patches = x.reshape(B, CIN, HL, P, HL, P).transpose(0, 2, 4, 1, 3, 5)
    patches = patches.reshape(N, KENC)
    we = enc_w.reshape(ZC, KENC).T
    wp = pre_w[:, :, 0, 0].T
    wpost = post_w[:, :, 0, 0].T
    wdec = dec_w[:, :, ::-1, ::-1].transpose(1, 0, 2, 3).reshape(ZC, CIN * P * P)
    db = jnp.repeat(dec_b, P * P).reshape(1, CIN * P * P)

    grid = N // ROWS
    full = lambda i: (0, 0)
    row_block = lambda shape: pl.BlockSpec(shape, lambda i: (i, 0))
    z_flat, zq_flat, rec_p = pl.pallas_call(
        _body,
        grid=(grid,),
        in_specs=[
            row_block((ROWS, KENC)),
            pl.BlockSpec((KENC, ZC), full),
            pl.BlockSpec((1, ZC), full),
            pl.BlockSpec((ZC, EMBED), full),
            pl.BlockSpec((1, EMBED), full),
            pl.BlockSpec((VOCAB, EMBED), full),
            pl.BlockSpec((EMBED, VOCAB), full),
            pl.BlockSpec((EMBED, ZC), full),
            pl.BlockSpec((1, ZC), full),
            pl.BlockSpec((ZC, CIN * P * P), full),
            pl.BlockSpec((1, CIN * P * P), full),
        ],
        out_specs=[
            row_block((ROWS, EMBED)),
            row_block((ROWS, EMBED)),
            row_block((ROWS, CIN * P * P)),
        ],
        out_shape=[
            jax.ShapeDtypeStruct((N, EMBED), jnp.float32),
            jax.ShapeDtypeStruct((N, EMBED), jnp.float32),
            jax.ShapeDtypeStruct((N, CIN * P * P), jnp.float32),
        ],
        compiler_params=pltpu.CompilerParams(
            dimension_semantics=("arbitrary",)),
    )(patches, we, enc_b.reshape(1, ZC), wp, pre_b.reshape(1, EMBED),
      codebook, codebook.T, wpost, post_b.reshape(1, ZC), wdec, db)

    z = z_flat.reshape(B, HL, HL, EMBED).transpose(0, 3, 1, 2)
    z_q = zq_flat.reshape(B, HL, HL, EMBED).transpose(0, 3, 1, 2)
    rec = rec_p.reshape(B, HL, HL, CIN, P, P).transpose(0, 3, 1, 4, 2, 5)
    rec = rec.reshape(B, CIN, HW, HW)
    return (z, z_q, rec)


# speedup vs baseline: 14.4211x; 1.5992x over previous
import jax
import jax.numpy as jnp
from jax.experimental import pallas as pl
from jax.experimental.pallas import tpu as pltpu

B, CIN, HW, P = 32, 3, 224, 16
ZC = 256
EMBED = 64
VOCAB = 8192
HL = HW // P
NPP = HL * HL
KENC = CIN * P * P

_DOT = jax.lax.Precision.DEFAULT
_EXACT = jax.lax.Precision.HIGHEST


def _body(x_ref, we_ref, eb_ref, wp_ref, pb_ref, cb_ref, cbt_ref,
          wpost_ref, postb_ref, wdec_ref, db_ref,
          z_ref, zq_ref, rec_ref, s2_ref):
    @pl.when(pl.program_id(0) == 0)
    def _():
        cbt = cbt_ref[...]
        s2_ref[...] = jnp.sum(cbt * cbt, axis=0, keepdims=True)

    xb = x_ref[0]
    p = xb.reshape(CIN, HL, P, HL, P).transpose(1, 3, 0, 2, 4)
    p = p.reshape(NPP, KENC)
    z0 = jax.lax.dot_general(p, we_ref[...], (((1,), (0,)), ((), ())),
                             precision=_DOT) + eb_ref[...]
    z = jax.lax.dot_general(z0, wp_ref[...], (((1,), (0,)), ((), ())),
                            precision=_DOT) + pb_ref[...]
    z_ref[0] = z.T
    s1 = jnp.sum(z * z, axis=1, keepdims=True)
    m = jax.lax.dot_general(z, cbt_ref[...], (((1,), (0,)), ((), ())),
                            precision=_DOT)
    dist = (s1 + s2_ref[...]) - 2.0 * m
    minval = jnp.min(dist, axis=1, keepdims=True)
    lane = jax.lax.broadcasted_iota(jnp.int32, dist.shape, 1)
    tok = jnp.min(jnp.where(dist == minval, lane, VOCAB), axis=1,
                  keepdims=True)
    oh = (lane == tok).astype(jnp.float32)
    zq = jax.lax.dot_general(oh, cb_ref[...], (((1,), (0,)), ((), ())),
                             precision=_EXACT)
    zq_ref[0] = zq.T
    d1 = jax.lax.dot_general(zq, wpost_ref[...], (((1,), (0,)), ((), ())),
                             precision=_DOT) + postb_ref[...]
    rec_p = jax.lax.dot_general(d1, wdec_ref[...], (((1,), (0,)), ((), ())),
                                precision=_DOT) + db_ref[...]
    rec = rec_p.reshape(HL, HL, CIN, P, P).transpose(2, 0, 3, 1, 4)
    rec_ref[0] = rec.reshape(CIN, HW, HW)


def kernel(x, enc_w, enc_b, pre_w, pre_b, codebook, post_w, post_b, dec_w, dec_b):
    we = enc_w.reshape(ZC, KENC).T
    wp = pre_w[:, :, 0, 0].T
    wpost = post_w[:, :, 0, 0].T
    wdec = dec_w[:, :, ::-1, ::-1].transpose(1, 0, 2, 3).reshape(ZC, CIN * P * P)
    db = jnp.repeat(dec_b, P * P).reshape(1, CIN * P * P)

    full = lambda b: (0, 0)
    z_out, zq_out, rec = pl.pallas_call(
        _body,
        grid=(B,),
        in_specs=[
            pl.BlockSpec((1, CIN, HW, HW), lambda b: (b, 0, 0, 0)),
            pl.BlockSpec((KENC, ZC), full),
            pl.BlockSpec((1, ZC), full),
            pl.BlockSpec((ZC, EMBED), full),
            pl.BlockSpec((1, EMBED), full),
            pl.BlockSpec((VOCAB, EMBED), full),
            pl.BlockSpec((EMBED, VOCAB), full),
            pl.BlockSpec((EMBED, ZC), full),
            pl.BlockSpec((1, ZC), full),
            pl.BlockSpec((ZC, CIN * P * P), full),
            pl.BlockSpec((1, CIN * P * P), full),
        ],
        out_specs=[
            pl.BlockSpec((1, EMBED, NPP), lambda b: (b, 0, 0)),
            pl.BlockSpec((1, EMBED, NPP), lambda b: (b, 0, 0)),
            pl.BlockSpec((1, CIN, HW, HW), lambda b: (b, 0, 0, 0)),
        ],
        out_shape=[
            jax.ShapeDtypeStruct((B, EMBED, NPP), jnp.float32),
            jax.ShapeDtypeStruct((B, EMBED, NPP), jnp.float32),
            jax.ShapeDtypeStruct((B, CIN, HW, HW), jnp.float32),
        ],
        scratch_shapes=[pltpu.VMEM((1, VOCAB), jnp.float32)],
        compiler_params=pltpu.CompilerParams(
            dimension_semantics=("arbitrary",)),
    )(x, we, enc_b.reshape(1, ZC), wp, pre_b.reshape(1, EMBED),
      codebook, codebook.T, wpost, post_b.reshape(1, ZC), wdec, db)

    z = z_out.reshape(B, EMBED, HL, HL)
    z_q = zq_out.reshape(B, EMBED, HL, HL)
    return (z, z_q, rec)


# speedup vs baseline: 17.9274x; 1.2431x over previous
import functools

import jax
import jax.numpy as jnp
from jax.experimental import pallas as pl
from jax.experimental.pallas import tpu as pltpu
from jax.experimental.pallas import tpu_sc as plsc

B, CIN, HW, P = 32, 3, 224, 16
ZC = 256
EMBED = 64
VOCAB = 8192
HL = HW // P
NPP = HL * HL
N = B * NPP
KENC = CIN * P * P
NW = 32
RPW = 256
CBW = 128

_DOT = jax.lax.Precision.DEFAULT


def _encode_body(x_ref, we_ref, eb_ref, wp_ref, pb_ref, cbt_ref,
                 z_ref, tok_ref, s2_ref):
    @pl.when(pl.program_id(0) == 0)
    def _():
        cbt = cbt_ref[...]
        s2_ref[...] = jnp.sum(cbt * cbt, axis=0, keepdims=True)

    xb = x_ref[0]
    p = xb.reshape(CIN, HL, P, HL, P).transpose(1, 3, 0, 2, 4)
    p = p.reshape(NPP, KENC)
    z0 = jax.lax.dot_general(p, we_ref[...], (((1,), (0,)), ((), ())),
                             precision=_DOT) + eb_ref[...]
    z = jax.lax.dot_general(z0, wp_ref[...], (((1,), (0,)), ((), ())),
                            precision=_DOT) + pb_ref[...]
    z_ref[0] = z.T
    s1 = jnp.sum(z * z, axis=1, keepdims=True)
    m = jax.lax.dot_general(z, cbt_ref[...], (((1,), (0,)), ((), ())),
                            precision=_DOT)
    dist = (s1 + s2_ref[...]) - 2.0 * m
    minval = jnp.min(dist, axis=1, keepdims=True)
    lane = jax.lax.broadcasted_iota(jnp.int32, dist.shape, 1)
    tok = jnp.min(jnp.where(dist == minval, lane, VOCAB), axis=1,
                  keepdims=True)
    tok_ref[0] = tok.T


def _decode_body(zq_ref, wpost_ref, postb_ref, wdec_ref, db_ref,
                 zqt_ref, rec_ref):
    zq = zq_ref[0, :NPP, :EMBED]
    zqt_ref[0] = zq.T
    d1 = jax.lax.dot_general(zq, wpost_ref[...], (((1,), (0,)), ((), ())),
                             precision=_DOT) + postb_ref[...]
    rec_p = jax.lax.dot_general(d1, wdec_ref[...], (((1,), (0,)), ((), ())),
                                precision=_DOT) + db_ref[...]
    rec = rec_p.reshape(HL, HL, CIN, P, P).transpose(2, 0, 3, 1, 4)
    rec_ref[0] = rec.reshape(CIN, HW, HW)


def _sc_gather_body(tok_hbm, table_hbm, out_hbm, idx_v, rows_v, sem0, sem1):
    wid = jax.lax.axis_index("s") * 2 + jax.lax.axis_index("c")
    pltpu.sync_copy(tok_hbm.at[pl.ds(wid * 2, 2)], idx_v)
    cp0 = pltpu.async_copy(table_hbm.at[idx_v.at[0]], rows_v.at[0], sem0)
    cp1 = pltpu.async_copy(table_hbm.at[idx_v.at[1]], rows_v.at[1], sem1)
    cp0.wait()
    cp1.wait()
    pltpu.sync_copy(rows_v.at[0], out_hbm.at[pl.ds(wid * RPW, 128)])
    pltpu.sync_copy(rows_v.at[1], out_hbm.at[pl.ds(wid * RPW + 128, 128)])


def _sc_gather(tok2d, table):
    gather = functools.partial(
        pl.kernel,
        mesh=plsc.VectorSubcoreMesh(core_axis_name="c", subcore_axis_name="s"),
        out_type=jax.ShapeDtypeStruct((NW * RPW, CBW), jnp.float32),
        scratch_types=[
            pltpu.VMEM((2, 128), jnp.int32),
            pltpu.VMEM((2, 128, CBW), jnp.float32),
            pltpu.SemaphoreType.DMA,
            pltpu.SemaphoreType.DMA,
        ],
    )(_sc_gather_body)
    return gather(tok2d, table)


def kernel(x, enc_w, enc_b, pre_w, pre_b, codebook, post_w, post_b, dec_w, dec_b):
    we = enc_w.reshape(ZC, KENC).T
    wp = pre_w[:, :, 0, 0].T
    wpost = post_w[:, :, 0, 0].T
    wdec = dec_w[:, :, ::-1, ::-1].transpose(1, 0, 2, 3).reshape(ZC, CIN * P * P)
    db = jnp.repeat(dec_b, P * P).reshape(1, CIN * P * P)

    full = lambda b: (0, 0)
    z_out, tok3 = pl.pallas_call(
        _encode_body,
        grid=(B,),
        in_specs=[
            pl.BlockSpec((1, CIN, HW, HW), lambda b: (b, 0, 0, 0)),
            pl.BlockSpec((KENC, ZC), full),
            pl.BlockSpec((1, ZC), full),
            pl.BlockSpec((ZC, EMBED), full),
            pl.BlockSpec((1, EMBED), full),
            pl.BlockSpec((EMBED, VOCAB), full),
        ],
        out_specs=[
            pl.BlockSpec((1, EMBED, NPP), lambda b: (b, 0, 0)),
            pl.BlockSpec((1, 1, NPP), lambda b: (b, 0, 0)),
        ],
        out_shape=[
            jax.ShapeDtypeStruct((B, EMBED, NPP), jnp.float32),
            jax.ShapeDtypeStruct((B, 1, NPP), jnp.int32),
        ],
        scratch_shapes=[pltpu.VMEM((1, VOCAB), jnp.float32)],
        compiler_params=pltpu.CompilerParams(
            dimension_semantics=("arbitrary",)),
    )(x, we, enc_b.reshape(1, ZC), wp, pre_b.reshape(1, EMBED), codebook.T)

    tok_pad = jnp.zeros((B, RPW), jnp.int32).at[:, :NPP].set(
        tok3.reshape(B, NPP))
    cb_pad = jnp.pad(codebook, ((0, 0), (0, CBW - EMBED)))
    zq_pad = _sc_gather(tok_pad.reshape(NW * RPW // 128, 128), cb_pad)
    zq_pad = zq_pad.reshape(B, RPW, CBW)

    zqt_out, rec = pl.pallas_call(
        _decode_body,
        grid=(B,),
        in_specs=[
            pl.BlockSpec((1, RPW, CBW), lambda b: (b, 0, 0)),
            pl.BlockSpec((EMBED, ZC), full),
            pl.BlockSpec((1, ZC), full),
            pl.BlockSpec((ZC, CIN * P * P), full),
            pl.BlockSpec((1, CIN * P * P), full),
        ],
        out_specs=[
            pl.BlockSpec((1, EMBED, NPP), lambda b: (b, 0, 0)),
            pl.BlockSpec((1, CIN, HW, HW), lambda b: (b, 0, 0, 0)),
        ],
        out_shape=[
            jax.ShapeDtypeStruct((B, EMBED, NPP), jnp.float32),
            jax.ShapeDtypeStruct((B, CIN, HW, HW), jnp.float32),
        ],
        compiler_params=pltpu.CompilerParams(
            dimension_semantics=("arbitrary",)),
    )(zq_pad, wpost, post_b.reshape(1, ZC), wdec, db)

    z = z_out.reshape(B, EMBED, HL, HL)
    z_q = zqt_out.reshape(B, EMBED, HL, HL)
    return (z, z_q, rec)


# speedup vs baseline: 18.7966x; 1.0485x over previous
import functools

import jax
import jax.numpy as jnp
from jax.experimental import pallas as pl
from jax.experimental.pallas import tpu as pltpu
from jax.experimental.pallas import tpu_sc as plsc

B, CIN, HW, P = 32, 3, 224, 16
ZC = 256
EMBED = 64
VOCAB = 8192
HL = HW // P
NPP = HL * HL
N = B * NPP
KENC = CIN * P * P
NW = 32
RPW = 256
CBW = 128

_DOT = jax.lax.Precision.DEFAULT


def _encode_body(x_ref, we_ref, eb_ref, wp_ref, pb_ref, cbt_ref,
                 z_ref, tok_ref, s2_ref):
    @pl.when(pl.program_id(0) == 0)
    def _():
        cbt = cbt_ref[...]
        s2_ref[...] = jnp.sum(cbt * cbt, axis=0, keepdims=True)

    xb = x_ref[0]
    p = xb.reshape(CIN, HL, P, HL, P).transpose(1, 3, 0, 2, 4)
    p = p.reshape(NPP, KENC)
    z0 = jax.lax.dot_general(p, we_ref[...], (((1,), (0,)), ((), ())),
                             precision=_DOT) + eb_ref[...]
    z = jax.lax.dot_general(z0, wp_ref[...], (((1,), (0,)), ((), ())),
                            precision=_DOT) + pb_ref[...]
    z_ref[0] = z.T
    s1 = jnp.sum(z * z, axis=1, keepdims=True)
    m = jax.lax.dot_general(z, cbt_ref[...], (((1,), (0,)), ((), ())),
                            precision=_DOT)
    dist = (s1 + s2_ref[...]) - 2.0 * m
    minval = jnp.min(dist, axis=1, keepdims=True)
    lane = jax.lax.broadcasted_iota(jnp.int32, dist.shape, 1)
    tok = jnp.min(jnp.where(dist == minval, lane, VOCAB), axis=1,
                  keepdims=True)
    tok_ref[0] = tok.T


def _decode_body(zq_ref, wpost_ref, postb_ref, wdec_ref, db_ref,
                 zqt_ref, rec_ref):
    zq = zq_ref[0, :NPP, :EMBED]
    zqt_ref[0] = zq.T
    d1 = jax.lax.dot_general(zq, wpost_ref[...], (((1,), (0,)), ((), ())),
                             precision=_DOT) + postb_ref[...]
    rec_p = jax.lax.dot_general(d1, wdec_ref[...], (((1,), (0,)), ((), ())),
                                precision=_DOT) + db_ref[...]
    rec = rec_p.reshape(HL, HL, CIN, P, P).transpose(2, 0, 3, 1, 4)
    rec_ref[0] = rec.reshape(CIN, HW, HW)


def _sc_gather_body(tok_hbm, table_hbm, out_hbm, idx_v, rows_v, sem0, sem1):
    wid = jax.lax.axis_index("s") * 2 + jax.lax.axis_index("c")
    pltpu.sync_copy(tok_hbm.at[pl.ds(wid * 2, 2)], idx_v)
    cp0 = pltpu.async_copy(table_hbm.at[idx_v.at[0]], rows_v.at[0], sem0)
    cp1 = pltpu.async_copy(table_hbm.at[idx_v.at[1]], rows_v.at[1], sem1)
    cp0.wait()
    cp1.wait()
    pltpu.sync_copy(rows_v.at[0], out_hbm.at[pl.ds(wid * RPW, 128)])
    pltpu.sync_copy(rows_v.at[1], out_hbm.at[pl.ds(wid * RPW + 128, 128)])


def _sc_gather_body_1(tok_hbm, table_hbm, out_hbm, idx_v, rows_v, sem0):
    wid = jax.lax.axis_index("s") * 2 + jax.lax.axis_index("c")
    pltpu.sync_copy(tok_hbm.at[pl.ds(wid, 1)], idx_v)
    pltpu.async_copy(table_hbm.at[idx_v.at[0]], rows_v.at[0], sem0).wait()
    pltpu.sync_copy(rows_v.at[0], out_hbm.at[pl.ds(wid * 128, 128)])


def _sc_gather(tok2d, table):
    nrow = tok2d.shape[0] * 128
    if nrow == NW * 128:
        body, nbuf, sems = _sc_gather_body_1, 1, [pltpu.SemaphoreType.DMA]
    else:
        body, nbuf, sems = (_sc_gather_body, 2,
                            [pltpu.SemaphoreType.DMA, pltpu.SemaphoreType.DMA])
    gather = functools.partial(
        pl.kernel,
        mesh=plsc.VectorSubcoreMesh(core_axis_name="c", subcore_axis_name="s"),
        out_type=jax.ShapeDtypeStruct((nrow, CBW), jnp.float32),
        scratch_types=[
            pltpu.VMEM((nbuf, 128), jnp.int32),
            pltpu.VMEM((nbuf, 128, CBW), jnp.float32),
        ] + sems,
    )(body)
    return gather(tok2d, table)


def _encode(xh, we, eb, wp, pb, cbt, nb):
    full = lambda b: (0, 0)
    return pl.pallas_call(
        _encode_body,
        grid=(nb,),
        in_specs=[
            pl.BlockSpec((1, CIN, HW, HW), lambda b: (b, 0, 0, 0)),
            pl.BlockSpec((KENC, ZC), full),
            pl.BlockSpec((1, ZC), full),
            pl.BlockSpec((ZC, EMBED), full),
            pl.BlockSpec((1, EMBED), full),
            pl.BlockSpec((EMBED, VOCAB), full),
        ],
        out_specs=[
            pl.BlockSpec((1, EMBED, NPP), lambda b: (b, 0, 0)),
            pl.BlockSpec((1, 1, NPP), lambda b: (b, 0, 0)),
        ],
        out_shape=[
            jax.ShapeDtypeStruct((nb, EMBED, NPP), jnp.float32),
            jax.ShapeDtypeStruct((nb, 1, NPP), jnp.int32),
        ],
        scratch_shapes=[pltpu.VMEM((1, VOCAB), jnp.float32)],
        compiler_params=pltpu.CompilerParams(
            dimension_semantics=("arbitrary",)),
    )(xh, we, eb, wp, pb, cbt)


def _decode(zq_pad, wpost, postb, wdec, db, nb):
    full = lambda b: (0, 0)
    return pl.pallas_call(
        _decode_body,
        grid=(nb,),
        in_specs=[
            pl.BlockSpec((1, RPW, CBW), lambda b: (b, 0, 0)),
            pl.BlockSpec((EMBED, ZC), full),
            pl.BlockSpec((1, ZC), full),
            pl.BlockSpec((ZC, CIN * P * P), full),
            pl.BlockSpec((1, CIN * P * P), full),
        ],
        out_specs=[
            pl.BlockSpec((1, EMBED, NPP), lambda b: (b, 0, 0)),
            pl.BlockSpec((1, CIN, HW, HW), lambda b: (b, 0, 0, 0)),
        ],
        out_shape=[
            jax.ShapeDtypeStruct((nb, EMBED, NPP), jnp.float32),
            jax.ShapeDtypeStruct((nb, CIN, HW, HW), jnp.float32),
        ],
        compiler_params=pltpu.CompilerParams(
            dimension_semantics=("arbitrary",)),
    )(zq_pad, wpost, postb, wdec, db)


def kernel(x, enc_w, enc_b, pre_w, pre_b, codebook, post_w, post_b, dec_w, dec_b):
    we = enc_w.reshape(ZC, KENC).T
    wp = pre_w[:, :, 0, 0].T
    wpost = post_w[:, :, 0, 0].T
    wdec = dec_w[:, :, ::-1, ::-1].transpose(1, 0, 2, 3).reshape(ZC, CIN * P * P)
    db = jnp.repeat(dec_b, P * P).reshape(1, CIN * P * P)
    eb = enc_b.reshape(1, ZC)
    pb = pre_b.reshape(1, EMBED)
    postb = post_b.reshape(1, ZC)
    cbt = codebook.T
    cb_pad = jnp.pad(codebook, ((0, 0), (0, CBW - EMBED)))
    filler = jnp.broadcast_to(jnp.arange(RPW, dtype=jnp.int32), (B // 2, RPW))

    HB = B // 2
    halves = []
    for h in range(2):
        xh = jax.lax.slice_in_dim(x, h * HB, (h + 1) * HB, axis=0)
        z_out, tok3 = _encode(xh, we, eb, wp, pb, cbt, HB)
        tok_pad = filler.at[:, :NPP].set(tok3.reshape(HB, NPP))
        zq_pad = _sc_gather(tok_pad.reshape(HB * RPW // 128, 128), cb_pad)
        halves.append((z_out, zq_pad.reshape(HB, RPW, CBW)))
    outs = []
    for h in range(2):
        z_out, zq_pad = halves[h]
        zqt_out, rec = _decode(zq_pad, wpost, postb, wdec, db, HB)
        outs.append((z_out, zqt_out, rec))

    z = jnp.concatenate([o[0] for o in outs]).reshape(B, EMBED, HL, HL)
    z_q = jnp.concatenate([o[1] for o in outs]).reshape(B, EMBED, HL, HL)
    rec = jnp.concatenate([o[2] for o in outs])
    return (z, z_q, rec)


# speedup vs baseline: 22.0762x; 1.1745x over previous
import functools

import jax
import jax.numpy as jnp
from jax.experimental import pallas as pl
from jax.experimental.pallas import tpu as pltpu
from jax.experimental.pallas import tpu_sc as plsc

B, CIN, HW, P = 32, 3, 224, 16
ZC = 256
EMBED = 64
VOCAB = 8192
HL = HW // P
NPP = HL * HL
N = B * NPP
KENC = CIN * P * P
NW = 32
RPW = 256
CBW = 128

_DOT = jax.lax.Precision.DEFAULT


def _encode_body(x_ref, we_ref, eb_ref, wp_ref, pb_ref, cbt_ref,
                 z_ref, tok_ref, s2_ref):
    @pl.when(pl.program_id(0) == 0)
    def _():
        cbt = cbt_ref[...]
        s2_ref[...] = jnp.sum(cbt * cbt, axis=0, keepdims=True)

    xb = x_ref[0]
    p = xb.reshape(CIN, HL, P, HL, P).transpose(1, 3, 0, 2, 4)
    p = p.reshape(NPP, KENC)
    z0 = jax.lax.dot_general(p, we_ref[...], (((1,), (0,)), ((), ())),
                             precision=_DOT) + eb_ref[...]
    z = jax.lax.dot_general(z0, wp_ref[...], (((1,), (0,)), ((), ())),
                            precision=_DOT) + pb_ref[...]
    z_ref[0] = z.T
    s1 = jnp.sum(z * z, axis=1, keepdims=True)
    m = jax.lax.dot_general(z, cbt_ref[...], (((1,), (0,)), ((), ())),
                            precision=_DOT)
    dist = (s1 + s2_ref[...]) - 2.0 * m
    minval = jnp.min(dist, axis=1, keepdims=True)
    lane = jax.lax.broadcasted_iota(jnp.int32, dist.shape, 1)
    tok = jnp.min(jnp.where(dist == minval, lane, VOCAB), axis=1,
                  keepdims=True)
    filler = jax.lax.broadcasted_iota(jnp.int32, (1, RPW - NPP), 1)
    tok_ref[0] = jnp.concatenate([tok.T, filler], axis=1)


def _encode_body_alias(x_ref, we_ref, eb_ref, wp_ref, pb_ref, cbt_ref,
                       zprev_ref, z_ref, tok_ref, s2_ref):
    del zprev_ref
    _encode_body(x_ref, we_ref, eb_ref, wp_ref, pb_ref, cbt_ref,
                 z_ref, tok_ref, s2_ref)


def _decode_body(zq_ref, wpost_ref, postb_ref, wdec_ref, db_ref,
                 zqt_ref, rec_ref):
    zq = zq_ref[0, :NPP, :EMBED]
    zqt_ref[0] = zq.T
    d1 = jax.lax.dot_general(zq, wpost_ref[...], (((1,), (0,)), ((), ())),
                             precision=_DOT) + postb_ref[...]
    rec_p = jax.lax.dot_general(d1, wdec_ref[...], (((1,), (0,)), ((), ())),
                                precision=_DOT) + db_ref[...]
    rec = rec_p.reshape(HL, HL, CIN, P, P).transpose(2, 0, 3, 1, 4)
    rec_ref[0] = rec.reshape(CIN, HW, HW)


def _decode_body_alias(zq_ref, wpost_ref, postb_ref, wdec_ref, db_ref,
                       zqtprev_ref, recprev_ref, zqt_ref, rec_ref):
    del zqtprev_ref, recprev_ref
    _decode_body(zq_ref, wpost_ref, postb_ref, wdec_ref, db_ref,
                 zqt_ref, rec_ref)


def _sc_gather_body(tok_hbm, table_hbm, out_hbm, idx_v, rows_v, sem0, sem1):
    wid = jax.lax.axis_index("s") * 2 + jax.lax.axis_index("c")
    pltpu.sync_copy(tok_hbm.at[pl.ds(wid * 2, 2)], idx_v)
    cp0 = pltpu.async_copy(table_hbm.at[idx_v.at[0]], rows_v.at[0], sem0)
    cp1 = pltpu.async_copy(table_hbm.at[idx_v.at[1]], rows_v.at[1], sem1)
    cp0.wait()
    cp1.wait()
    pltpu.sync_copy(rows_v.at[0], out_hbm.at[pl.ds(wid * RPW, 128)])
    pltpu.sync_copy(rows_v.at[1], out_hbm.at[pl.ds(wid * RPW + 128, 128)])


def _sc_gather_body_1(tok_hbm, table_hbm, out_hbm, idx_v, rows_v, sem0):
    wid = jax.lax.axis_index("s") * 2 + jax.lax.axis_index("c")
    pltpu.sync_copy(tok_hbm.at[pl.ds(wid, 1)], idx_v)
    pltpu.async_copy(table_hbm.at[idx_v.at[0]], rows_v.at[0], sem0).wait()
    pltpu.sync_copy(rows_v.at[0], out_hbm.at[pl.ds(wid * 128, 128)])


def _sc_gather(tok2d, table):
    nrow = tok2d.shape[0] * 128
    if nrow == NW * 128:
        body, nbuf, sems = _sc_gather_body_1, 1, [pltpu.SemaphoreType.DMA]
    else:
        body, nbuf, sems = (_sc_gather_body, 2,
                            [pltpu.SemaphoreType.DMA, pltpu.SemaphoreType.DMA])
    gather = functools.partial(
        pl.kernel,
        mesh=plsc.VectorSubcoreMesh(core_axis_name="c", subcore_axis_name="s"),
        out_type=jax.ShapeDtypeStruct((nrow, CBW), jnp.float32),
        scratch_types=[
            pltpu.VMEM((nbuf, 128), jnp.int32),
            pltpu.VMEM((nbuf, 128, CBW), jnp.float32),
        ] + sems,
    )(body)
    return gather(tok2d, table)


HB = B // 2


def _encode(x, we, eb, wp, pb, cbt, off, z_prev):
    full = lambda b: (0, 0)
    in_specs = [
        pl.BlockSpec((1, CIN, HW, HW), lambda b: (b + off, 0, 0, 0)),
        pl.BlockSpec((KENC, ZC), full),
        pl.BlockSpec((1, ZC), full),
        pl.BlockSpec((ZC, EMBED), full),
        pl.BlockSpec((1, EMBED), full),
        pl.BlockSpec((EMBED, VOCAB), full),
    ]
    args = [x, we, eb, wp, pb, cbt]
    body, aliases = _encode_body, {}
    if z_prev is not None:
        in_specs.append(pl.BlockSpec(memory_space=pltpu.MemorySpace.HBM))
        args.append(z_prev)
        body, aliases = _encode_body_alias, {6: 0}
    return pl.pallas_call(
        body,
        grid=(HB,),
        in_specs=in_specs,
        out_specs=[
            pl.BlockSpec((1, EMBED, NPP), lambda b: (b + off, 0, 0)),
            pl.BlockSpec((1, 1, RPW), lambda b: (b, 0, 0)),
        ],
        out_shape=[
            jax.ShapeDtypeStruct((B, EMBED, NPP), jnp.float32),
            jax.ShapeDtypeStruct((HB, 1, RPW), jnp.int32),
        ],
        scratch_shapes=[pltpu.VMEM((1, VOCAB), jnp.float32)],
        input_output_aliases=aliases,
        compiler_params=pltpu.CompilerParams(
            dimension_semantics=("arbitrary",)),
    )(*args)


def _decode(zq_pad, wpost, postb, wdec, db, off, prev):
    full = lambda b: (0, 0)
    in_specs = [
        pl.BlockSpec((1, RPW, CBW), lambda b: (b, 0, 0)),
        pl.BlockSpec((EMBED, ZC), full),
        pl.BlockSpec((1, ZC), full),
        pl.BlockSpec((ZC, CIN * P * P), full),
        pl.BlockSpec((1, CIN * P * P), full),
    ]
    args = [zq_pad, wpost, postb, wdec, db]
    body, aliases = _decode_body, {}
    if prev is not None:
        in_specs += [pl.BlockSpec(memory_space=pltpu.MemorySpace.HBM),
                     pl.BlockSpec(memory_space=pltpu.MemorySpace.HBM)]
        args += [prev[0], prev[1]]
        body, aliases = _decode_body_alias, {5: 0, 6: 1}
    return pl.pallas_call(
        body,
        grid=(HB,),
        in_specs=in_specs,
        out_specs=[
            pl.BlockSpec((1, EMBED, NPP), lambda b: (b + off, 0, 0)),
            pl.BlockSpec((1, CIN, HW, HW), lambda b: (b + off, 0, 0, 0)),
        ],
        out_shape=[
            jax.ShapeDtypeStruct((B, EMBED, NPP), jnp.float32),
            jax.ShapeDtypeStruct((B, CIN, HW, HW), jnp.float32),
        ],
        input_output_aliases=aliases,
        compiler_params=pltpu.CompilerParams(
            dimension_semantics=("arbitrary",)),
    )(*args)


def kernel(x, enc_w, enc_b, pre_w, pre_b, codebook, post_w, post_b, dec_w, dec_b):
    we = enc_w.reshape(ZC, KENC).T
    wp = pre_w[:, :, 0, 0].T
    wpost = post_w[:, :, 0, 0].T
    wdec = dec_w[:, :, ::-1, ::-1].transpose(1, 0, 2, 3).reshape(ZC, CIN * P * P)
    db = jnp.repeat(dec_b, P * P).reshape(1, CIN * P * P)
    eb = enc_b.reshape(1, ZC)
    pb = pre_b.reshape(1, EMBED)
    postb = post_b.reshape(1, ZC)
    cbt = codebook.T
    cb_pad = jnp.pad(codebook, ((0, 0), (0, CBW - EMBED)))

    z_out, tok0 = _encode(x, we, eb, wp, pb, cbt, 0, None)
    zq0 = _sc_gather(tok0.reshape(HB * RPW // 128, 128), cb_pad)
    z_out, tok1 = _encode(x, we, eb, wp, pb, cbt, HB, z_out)
    zq1 = _sc_gather(tok1.reshape(HB * RPW // 128, 128), cb_pad)
    zqt_out, rec = _decode(zq0.reshape(HB, RPW, CBW), wpost, postb, wdec, db,
                           0, None)
    zqt_out, rec = _decode(zq1.reshape(HB, RPW, CBW), wpost, postb, wdec, db,
                           HB, (zqt_out, rec))

    z = z_out.reshape(B, EMBED, HL, HL)
    z_q = zqt_out.reshape(B, EMBED, HL, HL)
    return (z, z_q, rec)
